# Initial kernel scaffold; baseline (speedup 1.0000x reference)
import jax
import jax.numpy as jnp
from jax.experimental import pallas as pl


def kernel(x, x1, edge_index, edge_index1, Wl0, bl0, Wr0, Wl1, bl1, Wr1, Vl0, cl0, Vr0, Vl1, cl1, Vr1, Wlin, blin):
    raise NotImplementedError("write your pallas kernel here")


# speedup vs baseline: 3.1444x; 3.1444x over previous
import jax
import jax.numpy as jnp
from jax import lax
from jax.experimental import pallas as pl
from jax.experimental.pallas import tpu as pltpu
from jax.experimental.pallas import tpu_sc as plsc

N = 10000
E = 160000
NC = 2
NS = 16
EB = 80
RPT = 624
TAIL = N - NS * RPT
E_PER = E // NS
NBLK = E_PER // EB

F32 = jnp.float32

_MESH = plsc.VectorSubcoreMesh(core_axis_name="c", subcore_axis_name="s",
                               num_cores=NC, num_subcores=NS)


def _zero_acc(zeros_hbm, acc_sh, s):
    pltpu.sync_copy(zeros_hbm, acc_sh.at[pl.ds(s * RPT, RPT)])

    @pl.when(s == 0)
    def _():
        pltpu.sync_copy(zeros_hbm.at[pl.ds(0, TAIL)],
                        acc_sh.at[pl.ds(NS * RPT, TAIL)])


def _drain_acc(acc_sh, out2d_hbm, s):
    pltpu.sync_copy(acc_sh.at[pl.ds(s * RPT, RPT)],
                    out2d_hbm.at[pl.ds(s * RPT, RPT)])

    @pl.when(s == 0)
    def _():
        pltpu.sync_copy(acc_sh.at[pl.ds(NS * RPT, TAIL)],
                        out2d_hbm.at[pl.ds(NS * RPT, TAIL)])


def _seg_sum(C):
    R = C // NC

    scratch = [
        pltpu.VMEM((EB,), jnp.int32),
        pltpu.VMEM((EB,), jnp.int32),
        pltpu.VMEM((EB, 128), F32),
        pltpu.VMEM_SHARED((N, 128), F32),
        pltpu.SemaphoreType.DMA,
    ]

    def body(x_hbm, src_hbm, dst_hbm, zeros_hbm, out_hbm,
             src_v, dst_v, rows_v, acc_sh, sem):
        c = lax.axis_index("c")
        s = lax.axis_index("s")

        for r in range(R):
            _zero_acc(zeros_hbm, acc_sh, s)
            plsc.subcore_barrier()

            for cc in range(NC):
                chunk = cc * R + r

                @pl.when(c == cc)
                def _(chunk=chunk):
                    def step(i, carry):
                        base = pl.multiple_of(s * E_PER + i * EB, 8)
                        pltpu.sync_copy(src_hbm.at[pl.ds(base, EB)], src_v)
                        pltpu.sync_copy(dst_hbm.at[pl.ds(base, EB)], dst_v)
                        pltpu.async_copy(x_hbm.at[chunk].at[src_v], rows_v,
                                         sem).wait()
                        pltpu.sync_copy(rows_v, acc_sh.at[dst_v], add=True)
                        return carry
                    lax.fori_loop(0, NBLK, step, 0)

            plsc.subcore_barrier()
            for cc in range(NC):
                @pl.when(c == cc)
                def _(cc=cc):
                    _drain_acc(acc_sh, out_hbm.at[cc * R + r], s)
            if r + 1 < R:
                plsc.subcore_barrier()

    return pl.kernel(body, out_type=jax.ShapeDtypeStruct((C, N, 128), F32),
                     mesh=_MESH, scratch_types=scratch)


NB0 = 62
NB1 = NBLK - NB0


def _count():
    scratch = [
        pltpu.VMEM((EB,), jnp.int32),
        pltpu.VMEM((EB, 128), F32),
        pltpu.VMEM_SHARED((N, 128), F32),
    ]

    def body(dst_hbm, zeros_hbm, ones_hbm, out_hbm, dst_v, ones_v, cnt_sh):
        c = lax.axis_index("c")
        s = lax.axis_index("s")

        pltpu.sync_copy(ones_hbm, ones_v)
        _zero_acc(zeros_hbm, cnt_sh, s)
        plsc.subcore_barrier()

        for cc, (b0, nb) in enumerate(((0, NB0), (NB0, NB1))):
            @pl.when(c == cc)
            def _(b0=b0, nb=nb):
                def step(i, carry):
                    base = pl.multiple_of(s * E_PER + (b0 + i) * EB, 8)
                    pltpu.sync_copy(dst_hbm.at[pl.ds(base, EB)], dst_v)
                    pltpu.sync_copy(ones_v, cnt_sh.at[dst_v], add=True)
                    return carry
                lax.fori_loop(0, nb, step, 0)

        plsc.subcore_barrier()
        for cc in range(NC):
            @pl.when(c == cc)
            def _(cc=cc):
                _drain_acc(cnt_sh, out_hbm.at[cc], s)

    return pl.kernel(body, out_type=jax.ShapeDtypeStruct((NC, N, 128), F32),
                     mesh=_MESH, scratch_types=scratch)


def _tc_layer1(acc, cnt2, x1, Wl, Wr, b):
    BN = 1000
    G = N // BN

    def body(acc_ref, cnt_ref, x_ref, wl_ref, wr_ref, b_ref, out_ref):
        cnt = cnt_ref[0][:, 0:1] + cnt_ref[1][:, 0:1]
        mc = jnp.maximum(cnt, 1.0)
        mean = jnp.concatenate([acc_ref[0], acc_ref[1]], axis=1) / mc
        h = jnp.dot(mean, wl_ref[...], preferred_element_type=F32)
        h = h + jnp.dot(x_ref[...], wr_ref[...], preferred_element_type=F32)
        h = jnp.maximum(h + b_ref[...], 0.0)
        for j in range(4):
            out_ref[j] = h[:, 128 * j:128 * (j + 1)]

    return pl.pallas_call(
        body,
        grid=(G,),
        in_specs=[
            pl.BlockSpec((2, BN, 128), lambda i: (0, i, 0)),
            pl.BlockSpec((2, BN, 128), lambda i: (0, i, 0)),
            pl.BlockSpec((BN, 256), lambda i: (i, 0)),
            pl.BlockSpec((256, 512), lambda i: (0, 0)),
            pl.BlockSpec((256, 512), lambda i: (0, 0)),
            pl.BlockSpec((1, 512), lambda i: (0, 0)),
        ],
        out_specs=pl.BlockSpec((4, BN, 128), lambda i: (0, i, 0)),
        out_shape=jax.ShapeDtypeStruct((4, N, 128), F32),
    )(acc, cnt2, x1, Wl, Wr, b.reshape(1, 512))


def _tc_layer2(acc, cnt2, h1, Wl, Wr, b, wlin, blin):
    BN = 1000
    G = N // BN

    def body(acc_ref, cnt_ref, h1_ref, wl_ref, wr_ref, b_ref, wlin_ref,
             blin_ref, out_ref, colsum):
        i = pl.program_id(0)
        cnt = cnt_ref[0][:, 0:1] + cnt_ref[1][:, 0:1]
        mc = jnp.maximum(cnt, 1.0)
        mean = jnp.concatenate([acc_ref[j] for j in range(4)], axis=1) / mc
        h1 = jnp.concatenate([h1_ref[j] for j in range(4)], axis=1)
        z = jnp.dot(mean, wl_ref[...], preferred_element_type=F32)
        z = z + jnp.dot(h1, wr_ref[...], preferred_element_type=F32)
        h2 = jnp.maximum(z + b_ref[...], 0.0)
        part = jnp.sum(h2, axis=0, keepdims=True)

        @pl.when(i == 0)
        def _():
            colsum[...] = part

        @pl.when(i > 0)
        def _():
            colsum[...] = colsum[...] + part

        @pl.when(i == G - 1)
        def _():
            out_ref[...] = (jnp.dot(colsum[...] / N, wlin_ref[...],
                                    preferred_element_type=F32)
                            + blin_ref[...])

    return pl.pallas_call(
        body,
        grid=(G,),
        in_specs=[
            pl.BlockSpec((4, BN, 128), lambda i: (0, i, 0)),
            pl.BlockSpec((2, BN, 128), lambda i: (0, i, 0)),
            pl.BlockSpec((4, BN, 128), lambda i: (0, i, 0)),
            pl.BlockSpec((512, 512), lambda i: (0, 0)),
            pl.BlockSpec((512, 512), lambda i: (0, 0)),
            pl.BlockSpec((1, 512), lambda i: (0, 0)),
            pl.BlockSpec((512, 1), lambda i: (0, 0)),
            pl.BlockSpec((1, 1), lambda i: (0, 0)),
        ],
        out_specs=pl.BlockSpec((1, 1), lambda i: (0, 0)),
        out_shape=jax.ShapeDtypeStruct((1, 1), F32),
        scratch_shapes=[pltpu.VMEM((1, 512), F32)],
    )(acc, cnt2, h1, Wl, Wr, b.reshape(1, 512), wlin, blin.reshape(1, 1))


def kernel(x, x1, edge_index, edge_index1, Wl0, bl0, Wr0, Wl1, bl1, Wr1,
           Vl0, cl0, Vr0, Vl1, cl1, Vr1, Wlin, blin):
    src = edge_index1[0].astype(jnp.int32)
    dst = edge_index1[1].astype(jnp.int32)

    x1c = x1.reshape(N, 2, 128).transpose(1, 0, 2)
    zeros = jnp.zeros((RPT, 128), F32)
    ones = jnp.ones((EB, 128), F32)

    cnt2 = _count()(dst, zeros, ones)
    acc1 = _seg_sum(2)(x1c, src, dst, zeros)
    h1c = _tc_layer1(acc1, cnt2, x1, Vl0, Vr0, cl0)
    acc2 = _seg_sum(4)(h1c, src, dst, zeros)
    return _tc_layer2(acc2, cnt2, h1c, Vl1, Vr1, cl1, Wlin, blin)


# speedup vs baseline: 5.6966x; 1.8117x over previous
import jax
import jax.numpy as jnp
from jax import lax
from jax.experimental import pallas as pl
from jax.experimental.pallas import tpu as pltpu
from jax.experimental.pallas import tpu_sc as plsc

N = 10000
E = 160000
NC = 2
NS = 16
EB = 80
RPT = 624
TAIL = N - NS * RPT
E_PER = E // NS
NBLK = E_PER // EB

F32 = jnp.float32

_MESH = plsc.VectorSubcoreMesh(core_axis_name="c", subcore_axis_name="s",
                               num_cores=NC, num_subcores=NS)


def _zero_acc(zeros_hbm, acc_sh, s):
    pltpu.sync_copy(zeros_hbm, acc_sh.at[pl.ds(s * RPT, RPT)])

    @pl.when(s == 0)
    def _():
        pltpu.sync_copy(zeros_hbm.at[pl.ds(0, TAIL)],
                        acc_sh.at[pl.ds(NS * RPT, TAIL)])


def _drain_acc(acc_sh, out2d_hbm, s):
    pltpu.sync_copy(acc_sh.at[pl.ds(s * RPT, RPT)],
                    out2d_hbm.at[pl.ds(s * RPT, RPT)])

    @pl.when(s == 0)
    def _():
        pltpu.sync_copy(acc_sh.at[pl.ds(NS * RPT, TAIL)],
                        out2d_hbm.at[pl.ds(NS * RPT, TAIL)])


def _seg_sum(C):
    R = C // NC

    scratch = [
        pltpu.VMEM((E_PER,), jnp.int32),
        pltpu.VMEM((EB,), jnp.int32),
        pltpu.VMEM((EB,), jnp.int32),
        pltpu.VMEM((EB, 128), F32),
        pltpu.VMEM((EB, 128), F32),
        pltpu.VMEM_SHARED((N, 128), F32),
        pltpu.SemaphoreType.DMA,
        pltpu.SemaphoreType.DMA,
        pltpu.SemaphoreType.DMA,
        pltpu.SemaphoreType.DMA,
    ]

    def body(x_hbm, src_hbm, dst_hbm, zeros_hbm, out_hbm,
             src_big, dst_v0, dst_v1, rows0, rows1, acc_sh,
             sg0, sg1, sd0, sd1):
        c = lax.axis_index("c")
        s = lax.axis_index("s")
        ebase = pl.multiple_of(s * E_PER, 8)
        pltpu.sync_copy(src_hbm.at[pl.ds(ebase, E_PER)], src_big)

        def issue(chunk, i, rows, dstv, sg, sd):
            pltpu.async_copy(
                x_hbm.at[chunk].at[src_big.at[pl.ds(i * EB, EB)]], rows, sg)
            pltpu.async_copy(
                dst_hbm.at[pl.ds(pl.multiple_of(ebase + i * EB, 8), EB)],
                dstv, sd)

        def wait(rows, dstv, sg, sd):
            pltpu.make_async_copy(x_hbm.at[0].at[pl.ds(0, EB)], rows,
                                  sg).wait()
            pltpu.make_async_copy(dst_hbm.at[pl.ds(0, EB)], dstv, sd).wait()

        for r in range(R):
            _zero_acc(zeros_hbm, acc_sh, s)
            plsc.subcore_barrier()

            for cc in range(NC):
                chunk = cc * R + r

                @pl.when(c == cc)
                def _(chunk=chunk):
                    issue(chunk, 0, rows0, dst_v0, sg0, sd0)

                    def pair(p, carry):
                        i0 = 2 * p
                        wait(rows0, dst_v0, sg0, sd0)
                        issue(chunk, i0 + 1, rows1, dst_v1, sg1, sd1)
                        pltpu.sync_copy(rows0, acc_sh.at[dst_v0], add=True)
                        wait(rows1, dst_v1, sg1, sd1)
                        issue(chunk, i0 + 2, rows0, dst_v0, sg0, sd0)
                        pltpu.sync_copy(rows1, acc_sh.at[dst_v1], add=True)
                        return carry

                    lax.fori_loop(0, (NBLK - 1) // 2, pair, 0)
                    wait(rows0, dst_v0, sg0, sd0)
                    pltpu.sync_copy(rows0, acc_sh.at[dst_v0], add=True)

            plsc.subcore_barrier()
            for cc in range(NC):
                @pl.when(c == cc)
                def _(cc=cc):
                    _drain_acc(acc_sh, out_hbm.at[cc * R + r], s)
            if r + 1 < R:
                plsc.subcore_barrier()

    return pl.kernel(body, out_type=jax.ShapeDtypeStruct((C, N, 128), F32),
                     mesh=_MESH, scratch_types=scratch)


NB0 = 62
NB1 = NBLK - NB0


def _count():
    scratch = [
        pltpu.VMEM((EB,), jnp.int32),
        pltpu.VMEM((EB,), jnp.int32),
        pltpu.VMEM((EB, 128), F32),
        pltpu.VMEM_SHARED((N, 128), F32),
        pltpu.SemaphoreType.DMA,
        pltpu.SemaphoreType.DMA,
    ]

    def body(dst_hbm, zeros_hbm, ones_hbm, out_hbm,
             dst_v0, dst_v1, ones_v, cnt_sh, sd0, sd1):
        c = lax.axis_index("c")
        s = lax.axis_index("s")
        ebase = pl.multiple_of(s * E_PER, 8)

        pltpu.sync_copy(ones_hbm, ones_v)
        _zero_acc(zeros_hbm, cnt_sh, s)
        plsc.subcore_barrier()

        def issue(i, dstv, sd):
            pltpu.async_copy(
                dst_hbm.at[pl.ds(pl.multiple_of(ebase + i * EB, 8), EB)],
                dstv, sd)

        def wait(dstv, sd):
            pltpu.make_async_copy(dst_hbm.at[pl.ds(0, EB)], dstv, sd).wait()

        for cc, (b0, nb) in enumerate(((0, NB0), (NB0, NB1))):
            @pl.when(c == cc)
            def _(b0=b0, nb=nb):
                issue(b0, dst_v0, sd0)
                npair = (nb - 1) // 2

                def pair(p, carry):
                    i0 = b0 + 2 * p
                    wait(dst_v0, sd0)
                    issue(i0 + 1, dst_v1, sd1)
                    pltpu.sync_copy(ones_v, cnt_sh.at[dst_v0], add=True)
                    wait(dst_v1, sd1)
                    issue(i0 + 2, dst_v0, sd0)
                    pltpu.sync_copy(ones_v, cnt_sh.at[dst_v1], add=True)
                    return carry

                lax.fori_loop(0, npair, pair, 0)
                wait(dst_v0, sd0)
                if nb % 2 == 0:
                    issue(b0 + nb - 1, dst_v1, sd1)
                pltpu.sync_copy(ones_v, cnt_sh.at[dst_v0], add=True)
                if nb % 2 == 0:
                    wait(dst_v1, sd1)
                    pltpu.sync_copy(ones_v, cnt_sh.at[dst_v1], add=True)

        plsc.subcore_barrier()
        for cc in range(NC):
            @pl.when(c == cc)
            def _(cc=cc):
                _drain_acc(cnt_sh, out_hbm.at[cc], s)

    return pl.kernel(body, out_type=jax.ShapeDtypeStruct((NC, N, 128), F32),
                     mesh=_MESH, scratch_types=scratch)


def _tc_layer1(acc, cnt2, x1, Wl, Wr, b):
    BN = 1000
    G = N // BN

    def body(acc_ref, cnt_ref, x_ref, wl_ref, wr_ref, b_ref, out_ref):
        cnt = cnt_ref[0][:, 0:1] + cnt_ref[1][:, 0:1]
        mc = jnp.maximum(cnt, 1.0)
        mean = jnp.concatenate([acc_ref[0], acc_ref[1]], axis=1) / mc
        h = jnp.dot(mean, wl_ref[...], preferred_element_type=F32)
        h = h + jnp.dot(x_ref[...], wr_ref[...], preferred_element_type=F32)
        h = jnp.maximum(h + b_ref[...], 0.0)
        for j in range(4):
            out_ref[j] = h[:, 128 * j:128 * (j + 1)]

    return pl.pallas_call(
        body,
        grid=(G,),
        in_specs=[
            pl.BlockSpec((2, BN, 128), lambda i: (0, i, 0)),
            pl.BlockSpec((2, BN, 128), lambda i: (0, i, 0)),
            pl.BlockSpec((BN, 256), lambda i: (i, 0)),
            pl.BlockSpec((256, 512), lambda i: (0, 0)),
            pl.BlockSpec((256, 512), lambda i: (0, 0)),
            pl.BlockSpec((1, 512), lambda i: (0, 0)),
        ],
        out_specs=pl.BlockSpec((4, BN, 128), lambda i: (0, i, 0)),
        out_shape=jax.ShapeDtypeStruct((4, N, 128), F32),
    )(acc, cnt2, x1, Wl, Wr, b.reshape(1, 512))


def _tc_layer2(acc, cnt2, h1, Wl, Wr, b, wlin, blin):
    BN = 1000
    G = N // BN

    def body(acc_ref, cnt_ref, h1_ref, wl_ref, wr_ref, b_ref, wlin_ref,
             blin_ref, out_ref, colsum):
        i = pl.program_id(0)
        cnt = cnt_ref[0][:, 0:1] + cnt_ref[1][:, 0:1]
        mc = jnp.maximum(cnt, 1.0)
        mean = jnp.concatenate([acc_ref[j] for j in range(4)], axis=1) / mc
        h1 = jnp.concatenate([h1_ref[j] for j in range(4)], axis=1)
        z = jnp.dot(mean, wl_ref[...], preferred_element_type=F32)
        z = z + jnp.dot(h1, wr_ref[...], preferred_element_type=F32)
        h2 = jnp.maximum(z + b_ref[...], 0.0)
        part = jnp.sum(h2, axis=0, keepdims=True)

        @pl.when(i == 0)
        def _():
            colsum[...] = part

        @pl.when(i > 0)
        def _():
            colsum[...] = colsum[...] + part

        @pl.when(i == G - 1)
        def _():
            out_ref[...] = (jnp.dot(colsum[...] / N, wlin_ref[...],
                                    preferred_element_type=F32)
                            + blin_ref[...])

    return pl.pallas_call(
        body,
        grid=(G,),
        in_specs=[
            pl.BlockSpec((4, BN, 128), lambda i: (0, i, 0)),
            pl.BlockSpec((2, BN, 128), lambda i: (0, i, 0)),
            pl.BlockSpec((4, BN, 128), lambda i: (0, i, 0)),
            pl.BlockSpec((512, 512), lambda i: (0, 0)),
            pl.BlockSpec((512, 512), lambda i: (0, 0)),
            pl.BlockSpec((1, 512), lambda i: (0, 0)),
            pl.BlockSpec((512, 1), lambda i: (0, 0)),
            pl.BlockSpec((1, 1), lambda i: (0, 0)),
        ],
        out_specs=pl.BlockSpec((1, 1), lambda i: (0, 0)),
        out_shape=jax.ShapeDtypeStruct((1, 1), F32),
        scratch_shapes=[pltpu.VMEM((1, 512), F32)],
    )(acc, cnt2, h1, Wl, Wr, b.reshape(1, 512), wlin, blin.reshape(1, 1))


def kernel(x, x1, edge_index, edge_index1, Wl0, bl0, Wr0, Wl1, bl1, Wr1,
           Vl0, cl0, Vr0, Vl1, cl1, Vr1, Wlin, blin):
    src = edge_index1[0].astype(jnp.int32)
    dst = edge_index1[1].astype(jnp.int32)

    x1c = x1.reshape(N, 2, 128).transpose(1, 0, 2)
    zeros = jnp.zeros((RPT, 128), F32)
    ones = jnp.ones((EB, 128), F32)

    cnt2 = _count()(dst, zeros, ones)
    acc1 = _seg_sum(2)(x1c, src, dst, zeros)
    h1c = _tc_layer1(acc1, cnt2, x1, Vl0, Vr0, cl0)
    acc2 = _seg_sum(4)(h1c, src, dst, zeros)
    return _tc_layer2(acc2, cnt2, h1c, Vl1, Vr1, cl1, Wlin, blin)
